# baseline (speedup 1.0000x reference)
import jax
import jax.numpy as jnp
import numpy as np
from jax import lax
from jax.experimental import pallas as pl
from jax.experimental.pallas import tpu as pltpu
from jax.experimental.pallas import tpu_sc as plsc

BATCH = 512
N_LOGITS = 33416
Y = 1024
X = 64

_NVEC = (np.arange(Y) % 63) + 2
_START = np.concatenate([[0], np.cumsum(_NVEC)])

_NPIECES = Y // 128
_YW = 128
_LO = [int(_START[_YW * k]) for k in range(_NPIECES)]
_HI = [int(_START[_YW * (k + 1)]) for k in range(_NPIECES)]
_W0 = [128 * (_LO[k] // 128) for k in range(_NPIECES)]
_FULL_TILES = 128 * (N_LOGITS // 128)
_WIDTH = [128 * ((_HI[k] - _W0[k] + 127) // 128) for k in range(_NPIECES)]
_WIDTH[-1] = _FULL_TILES - _W0[-1]
_STAGE_COLS = 128 * ((max(_WIDTH) + 64 + 127) // 128)
_STRAG_DST = _FULL_TILES - _W0[-1]

NC = 2
NS = 16
NW = NC * NS
GROUPS = BATCH // 8
GROUPS_PER_WORKER = GROUPS // NW


def _sc_body(logits_hbm, start_hbm, nvec_hbm, out_hbm, stage_a, stage_b,
             start_v, nvec_v, piece_v0, piece_v1, strag_v, sem0, sem1,
             sem_stage):
  c = lax.axis_index("c")
  s = lax.axis_index("s")
  wid = s * NC + c

  pltpu.sync_copy(start_hbm, start_v)
  pltpu.sync_copy(nvec_hbm, nvec_v)
  ninf = jnp.float32(-jnp.inf)

  def group_body(gi, _):
    g = wid + NW * gi
    pltpu.async_copy(
        logits_hbm.at[pl.ds(g * 8, 8), pl.ds(_W0[0], _WIDTH[0])],
        stage_a.at[:, pl.ds(0, _WIDTH[0])], sem_stage)
    for k in range(_NPIECES):
      w0 = _W0[k]
      width = _WIDTH[k]
      stage_v = stage_a if k % 2 == 0 else stage_b
      pltpu.make_async_copy(
          logits_hbm.at[pl.ds(0, 8), pl.ds(0, width)],
          stage_v.at[:, pl.ds(0, width)], sem_stage).wait()
      if k < _NPIECES - 1:
        nxt = stage_b if k % 2 == 0 else stage_a
        pltpu.async_copy(
            logits_hbm.at[pl.ds(g * 8, 8),
                          pl.ds(_W0[k + 1], _WIDTH[k + 1])],
            nxt.at[:, pl.ds(0, _WIDTH[k + 1])], sem_stage)
      if k == _NPIECES - 1:
        pltpu.sync_copy(
            logits_hbm.at[pl.ds(g * 8, 8), pl.ds(_FULL_TILES, 8)], strag_v)
        lane = lax.iota(jnp.int32, 16)

        def patch(sp, _):
          sp16 = jnp.broadcast_to(sp, (16,))
          v = plsc.load_gather(strag_v, [sp16, lane & 7])
          plsc.store_scatter(stage_v, [sp16, _STRAG_DST + (lane & 7)], v,
                             mask=lane < 8)
          return _
        lax.fori_loop(0, 8, patch, None)

      def wait_piece(piece_v, sem):
        pltpu.make_async_copy(
            piece_v, out_hbm.at[0, :, pl.ds(0, _YW)], sem).wait()

      def process(piece_v, sem, sample, wait):
        b = g * 8 + sample
        if wait == 'maybe':
          @pl.when(gi > 0)
          def _w():
            wait_piece(piece_v, sem)
        elif wait:
          wait_piece(piece_v, sem)

        def yc_body(yc, _):
          ybase = _YW * k + 16 * yc
          svec = start_v[pl.ds(ybase, 16)] - jnp.int32(w0)
          nvec = nvec_v[pl.ds(ybase, 16)]

          def xg_body(xg, _):
            for xi in range(8):
              x = xg * 8 + xi
              v = plsc.load_gather(stage_v,
                                   [jnp.broadcast_to(sample, (16,)),
                                    svec + x])
              v = jnp.where(x < nvec, v, ninf)
              piece_v[x, pl.ds(yc * 16, 16)] = v
            return _

          lax.fori_loop(0, 8, xg_body, None)
          return _

        lax.fori_loop(0, 8, yc_body, None)
        pltpu.async_copy(piece_v, out_hbm.at[b, :, pl.ds(_YW * k, _YW)], sem)

      def pair_body(t, _):
        process(piece_v0, sem0, 2 * t, True)
        process(piece_v1, sem1, 2 * t + 1, True)
        return _

      w01 = 'maybe' if k == 0 else True
      process(piece_v0, sem0, 0, w01)
      process(piece_v1, sem1, 1, w01)
      lax.fori_loop(1, 4, pair_body, None)
    return _

  lax.fori_loop(0, GROUPS_PER_WORKER, group_body, None)
  for pv, sm in ((piece_v0, sem0), (piece_v1, sem1)):
    pltpu.make_async_copy(pv, out_hbm.at[0, :, pl.ds(0, _YW)], sm).wait()


def kernel(logits):
  mesh = plsc.VectorSubcoreMesh(core_axis_name="c", subcore_axis_name="s")
  run = pl.kernel(
      _sc_body,
      out_type=jax.ShapeDtypeStruct((BATCH, X, Y), jnp.float32),
      mesh=mesh,
      scratch_types=[
          pltpu.VMEM((8, _STAGE_COLS), jnp.float32),
          pltpu.VMEM((8, _STAGE_COLS), jnp.float32),
          pltpu.VMEM((Y, ), jnp.int32),
          pltpu.VMEM((Y, ), jnp.int32),
          pltpu.VMEM((X, _YW), jnp.float32),
          pltpu.VMEM((X, _YW), jnp.float32),
          pltpu.VMEM((8, 8), jnp.float32),
          pltpu.SemaphoreType.DMA,
          pltpu.SemaphoreType.DMA,
          pltpu.SemaphoreType.DMA,
      ],
      compiler_params=pltpu.CompilerParams(
          use_tc_tiling_on_sc=True, needs_layout_passes=False),
  )
  out = run(logits, jnp.asarray(_START[:Y].astype(np.int32)),
            jnp.asarray(_NVEC.astype(np.int32)))
  return out.transpose(0, 2, 1)


# speedup vs baseline: 1.8844x; 1.8844x over previous
import jax
import jax.numpy as jnp
import numpy as np
from jax import lax
from jax.experimental import pallas as pl
from jax.experimental.pallas import tpu as pltpu
from jax.experimental.pallas import tpu_sc as plsc

BATCH = 512
N_LOGITS = 33416
Y = 1024
X = 64

_NVEC = (np.arange(Y) % 63) + 2
_START = np.concatenate([[0], np.cumsum(_NVEC)])

_NPIECES = Y // 128
_YW = 128
_LO = [int(_START[_YW * k]) for k in range(_NPIECES)]
_HI = [int(_START[_YW * (k + 1)]) for k in range(_NPIECES)]
_W0 = [128 * (_LO[k] // 128) for k in range(_NPIECES)]
_FULL_TILES = 128 * (N_LOGITS // 128)
_WIDTH = [128 * ((_HI[k] - _W0[k] + 127) // 128) for k in range(_NPIECES)]
_WIDTH[-1] = _FULL_TILES - _W0[-1]
_STAGE_COLS = 128 * ((max(_WIDTH) + 64 + 127) // 128)
_STRAG_DST = _FULL_TILES - _W0[-1]

NC = 2
NS = 16
NW = NC * NS
GROUPS = BATCH // 8
GROUPS_PER_WORKER = GROUPS // NW


def _sc_body(logits_hbm, start_hbm, nvec_hbm, out_hbm, stage_a, stage_b,
             start_v, nvec_v, piece_v0, piece_v1, strag_v, sem0, sem1,
             sem_stage):
  c = lax.axis_index("c")
  s = lax.axis_index("s")
  wid = s * NC + c

  pltpu.sync_copy(start_hbm, start_v)
  pltpu.sync_copy(nvec_hbm, nvec_v)
  ninf = jnp.float32(-jnp.inf)

  def group_body(gi, _):
    g = wid + NW * gi
    pltpu.async_copy(
        logits_hbm.at[pl.ds(g * 8, 8), pl.ds(_W0[0], _WIDTH[0])],
        stage_a.at[:, pl.ds(0, _WIDTH[0])], sem_stage)
    for k in range(_NPIECES):
      w0 = _W0[k]
      width = _WIDTH[k]
      stage_v = stage_a if k % 2 == 0 else stage_b
      pltpu.make_async_copy(
          logits_hbm.at[pl.ds(0, 8), pl.ds(0, width)],
          stage_v.at[:, pl.ds(0, width)], sem_stage).wait()
      if k < _NPIECES - 1:
        nxt = stage_b if k % 2 == 0 else stage_a
        pltpu.async_copy(
            logits_hbm.at[pl.ds(g * 8, 8),
                          pl.ds(_W0[k + 1], _WIDTH[k + 1])],
            nxt.at[:, pl.ds(0, _WIDTH[k + 1])], sem_stage)
      if k == _NPIECES - 1:
        pltpu.sync_copy(
            logits_hbm.at[pl.ds(g * 8, 8), pl.ds(_FULL_TILES, 8)], strag_v)
        lane = lax.iota(jnp.int32, 16)

        def patch(sp, _):
          sp16 = jnp.broadcast_to(sp, (16,))
          v = plsc.load_gather(strag_v, [sp16, lane & 7])
          plsc.store_scatter(stage_v, [sp16, _STRAG_DST + (lane & 7)], v,
                             mask=lane < 8)
          return _
        lax.fori_loop(0, 8, patch, None)

      def wait_piece(piece_v, sem):
        pltpu.make_async_copy(
            piece_v, out_hbm.at[0, :, pl.ds(0, _YW)], sem).wait()

      def process(piece_v, sem, sample, wait):
        b = g * 8 + sample
        if wait == 'maybe':
          @pl.when(gi > 0)
          def _w():
            wait_piece(piece_v, sem)
        elif wait:
          wait_piece(piece_v, sem)

        def yc_body(yc, _):
          ybase = _YW * k + 16 * yc
          svec = start_v[pl.ds(ybase, 16)] - jnp.int32(w0)
          nvec = nvec_v[pl.ds(ybase, 16)]

          sample16 = jnp.broadcast_to(sample, (16,))

          def xg_body(xg, _):
            xs = [xg * 8 + xi for xi in range(8)]
            vs = [plsc.load_gather(stage_v, [sample16, svec + x])
                  for x in xs]
            ms = [x < nvec for x in xs]
            for xi in range(8):
              piece_v[xs[xi], pl.ds(yc * 16, 16)] = jnp.where(
                  ms[xi], vs[xi], ninf)
            return _

          lax.fori_loop(0, 8, xg_body, None)
          return _

        lax.fori_loop(0, 8, yc_body, None)
        for xg in range(8):
          pltpu.async_copy(
              piece_v.at[pl.ds(8 * xg, 8), :],
              out_hbm.at[b, pl.ds(8 * xg, 8), pl.ds(_YW * k, _YW)], sem)

      def pair_body(t, _):
        process(piece_v0, sem0, 2 * t, True)
        process(piece_v1, sem1, 2 * t + 1, True)
        return _

      w01 = 'maybe' if k == 0 else True
      process(piece_v0, sem0, 0, w01)
      process(piece_v1, sem1, 1, w01)
      lax.fori_loop(1, 4, pair_body, None)
    return _

  lax.fori_loop(0, GROUPS_PER_WORKER, group_body, None)
  for pv, sm in ((piece_v0, sem0), (piece_v1, sem1)):
    pltpu.make_async_copy(pv, out_hbm.at[0, :, pl.ds(0, _YW)], sm).wait()


def kernel(logits):
  mesh = plsc.VectorSubcoreMesh(core_axis_name="c", subcore_axis_name="s")
  run = pl.kernel(
      _sc_body,
      out_type=jax.ShapeDtypeStruct((BATCH, X, Y), jnp.float32),
      mesh=mesh,
      scratch_types=[
          pltpu.VMEM((8, _STAGE_COLS), jnp.float32),
          pltpu.VMEM((8, _STAGE_COLS), jnp.float32),
          pltpu.VMEM((Y, ), jnp.int32),
          pltpu.VMEM((Y, ), jnp.int32),
          pltpu.VMEM((X, _YW), jnp.float32),
          pltpu.VMEM((X, _YW), jnp.float32),
          pltpu.VMEM((8, 8), jnp.float32),
          pltpu.SemaphoreType.DMA,
          pltpu.SemaphoreType.DMA,
          pltpu.SemaphoreType.DMA,
      ],
      compiler_params=pltpu.CompilerParams(
          use_tc_tiling_on_sc=True, needs_layout_passes=False),
  )
  out = run(logits, jnp.asarray(_START[:Y].astype(np.int32)),
            jnp.asarray(_NVEC.astype(np.int32)))
  return out.transpose(0, 2, 1)
